# baseline (speedup 1.0000x reference)
import functools

import jax
import jax.numpy as jnp
from jax import lax
from jax.experimental import pallas as pl
from jax.experimental.pallas import tpu as pltpu
from jax.experimental.pallas import tpu_sc as plsc

B = 16384
D = 128
V = 100
VPAD = 128
P = 50

NC, NS = 2, 16
NW = NC * NS

B_SC = 8192
BPW = B_SC // NW

BLK = 2048
NB = B // BLK
NSCB = B_SC // BLK


@functools.cache
def _make_sc_gather():
    mesh = plsc.VectorSubcoreMesh(core_axis_name="c", subcore_axis_name="s",
                                  num_cores=NC, num_subcores=NS)

    @functools.partial(
        pl.kernel,
        out_type=jax.ShapeDtypeStruct((B_SC, D), jnp.float32),
        mesh=mesh,
        scratch_types=[
            pltpu.VMEM((BPW,), jnp.int32),
            pltpu.VMEM((BPW, D), jnp.float32),
            pltpu.SemaphoreType.DMA,
        ],
    )
    def _sc_gather(table_hbm, idx_hbm, out_hbm, idx_v, rows_v, sem):
        wid = lax.axis_index("s") * NC + lax.axis_index("c")
        base = wid * BPW
        pltpu.sync_copy(idx_hbm.at[pl.ds(base, BPW)], idx_v)
        pltpu.async_copy(table_hbm.at[idx_v], rows_v, sem).wait()
        pltpu.sync_copy(rows_v, out_hbm.at[pl.ds(base, BPW)])

    return _sc_gather


def _fuse_into_scratch(i, wproj_ref, wprop_ref, bprop_ref, bproj_ref, wf_ref,
                       bias_ref):
    @pl.when(i == 0)
    def _():
        wbot = wproj_ref[D:, :]
        wf_ref[...] = wprop_ref[...] @ wbot
        bias_ref[...] = bprop_ref[...] @ wbot + bproj_ref[...]


def _onehot_body(wproj_ref, wtop_ref, wprop_ref, bprop_ref, bproj_ref,
                 table_ref, ids_ref, prop_ref, out_ref, wf_ref, bias_ref):
    i = pl.program_id(0)
    _fuse_into_scratch(i, wproj_ref, wprop_ref, bprop_ref, bproj_ref, wf_ref,
                       bias_ref)
    common = prop_ref[...] @ wf_ref[...] + bias_ref[...]
    iota_v = lax.broadcasted_iota(jnp.int32, (V, 1), 0)
    oh_t = (ids_ref[0] == iota_v).astype(jnp.bfloat16)
    g_blk = lax.dot_general(oh_t, table_ref[...],
                            (((0,), (0,)), ((), ())),
                            preferred_element_type=jnp.float32)
    g_blk = g_blk.astype(jnp.bfloat16)
    out_ref[...] = lax.dot_general(g_blk, wtop_ref[...],
                                   (((1,), (0,)), ((), ())),
                                   preferred_element_type=jnp.float32) + common


def _gathered_body(prev_ref, wproj_ref, wtop_ref, wprop_ref, bprop_ref,
                   bproj_ref, g_ref, prop_ref, out_ref, wf_ref, bias_ref):
    del prev_ref
    i = pl.program_id(0)
    _fuse_into_scratch(i, wproj_ref, wprop_ref, bprop_ref, bproj_ref, wf_ref,
                       bias_ref)
    common = prop_ref[...] @ wf_ref[...] + bias_ref[...]
    g_bf = g_ref[...].astype(jnp.bfloat16)
    out_ref[...] = lax.dot_general(g_bf, wtop_ref[...],
                                   (((1,), (0,)), ((), ())),
                                   preferred_element_type=jnp.float32) + common


PPAD = 128

_WSPECS = [
    pl.BlockSpec((2 * D, D), lambda i: (0, 0)),
    pl.BlockSpec((D, D), lambda i: (0, 0)),
    pl.BlockSpec((PPAD, D), lambda i: (0, 0)),
    pl.BlockSpec((1, D), lambda i: (0, 0)),
    pl.BlockSpec((1, D), lambda i: (0, 0)),
]
_SCRATCH = [
    pltpu.VMEM((PPAD, D), jnp.float32),
    pltpu.VMEM((1, D), jnp.float32),
]
_OUT_SHAPE = jax.ShapeDtypeStruct((B, D), jnp.float32)


def _combine_onehot(w_proj, wtop_bf, w_prop, b_prop, b_proj, table_bf, ids3,
                    prop):
    return pl.pallas_call(
        _onehot_body,
        grid=(NB - NSCB,),
        in_specs=_WSPECS + [
            pl.BlockSpec((V, D), lambda i: (0, 0)),
            pl.BlockSpec((1, 1, BLK), lambda i: (i + NSCB, 0, 0)),
            pl.BlockSpec((BLK, PPAD), lambda i: (i + NSCB, 0)),
        ],
        out_specs=pl.BlockSpec((BLK, D), lambda i: (i + NSCB, 0)),
        out_shape=_OUT_SHAPE,
        scratch_shapes=_SCRATCH,
    )(w_proj, wtop_bf, w_prop, b_prop, b_proj, table_bf, ids3, prop)


def _combine_gathered(prev, w_proj, wtop_bf, w_prop, b_prop, b_proj, g, prop):
    return pl.pallas_call(
        _gathered_body,
        grid=(NSCB,),
        in_specs=[pl.BlockSpec(memory_space=pl.ANY)] + _WSPECS + [
            pl.BlockSpec((BLK, D), lambda i: (i, 0)),
            pl.BlockSpec((BLK, PPAD), lambda i: (i, 0)),
        ],
        out_specs=pl.BlockSpec((BLK, D), lambda i: (i, 0)),
        out_shape=_OUT_SHAPE,
        input_output_aliases={0: 0},
        scratch_shapes=_SCRATCH,
    )(prev, w_proj, wtop_bf, w_prop, b_prop, b_proj, g, prop)


def kernel(event_type_ids, prop_vectors, event_type_table, W_prop, b_prop,
           W_proj, b_proj):
    ids = event_type_ids.astype(jnp.int32)
    g = _make_sc_gather()(event_type_table, ids)
    bprop2 = b_prop.reshape(1, D)
    bproj2 = b_proj.reshape(1, D)
    ids3 = ids.reshape(NB, 1, BLK)
    wtop_bf = W_proj[:D].astype(jnp.bfloat16)
    table_bf = event_type_table.astype(jnp.bfloat16)
    prop_pad = jnp.pad(prop_vectors, ((0, 0), (0, PPAD - P)))
    wprop_pad = jnp.pad(W_prop, ((0, PPAD - P), (0, 0)))
    out = _combine_onehot(W_proj, wtop_bf, wprop_pad, bprop2, bproj2, table_bf,
                          ids3, prop_pad)
    return _combine_gathered(out, W_proj, wtop_bf, wprop_pad, bprop2, bproj2,
                             g, prop_pad)


# speedup vs baseline: 1.1951x; 1.1951x over previous
import functools

import jax
import jax.numpy as jnp
from jax import lax
from jax.experimental import pallas as pl
from jax.experimental.pallas import tpu as pltpu
from jax.experimental.pallas import tpu_sc as plsc

B = 16384
D = 128
V = 100
VPAD = 128
P = 50

NC, NS = 2, 16
NW = NC * NS

B_SC = 8192
BPW = B_SC // NW

BLK = 2048
NB = B // BLK
NSCB = B_SC // BLK


@functools.cache
def _make_sc_gather():
    mesh = plsc.VectorSubcoreMesh(core_axis_name="c", subcore_axis_name="s",
                                  num_cores=NC, num_subcores=NS)

    @functools.partial(
        pl.kernel,
        out_type=jax.ShapeDtypeStruct((B_SC, D), jnp.float32),
        mesh=mesh,
        scratch_types=[
            pltpu.VMEM((BPW,), jnp.int32),
            pltpu.VMEM((BPW, D), jnp.float32),
            pltpu.SemaphoreType.DMA,
        ],
    )
    def _sc_gather(table_hbm, idx_hbm, out_hbm, idx_v, rows_v, sem):
        wid = lax.axis_index("s") * NC + lax.axis_index("c")
        base = wid * BPW
        pltpu.sync_copy(idx_hbm.at[pl.ds(base, BPW)], idx_v)
        pltpu.async_copy(table_hbm.at[idx_v], rows_v, sem).wait()
        pltpu.sync_copy(rows_v, out_hbm.at[pl.ds(base, BPW)])

    return _sc_gather


def _fuse_into_scratch(i, wproj_ref, wprop_ref, bprop_ref, bproj_ref, wf_ref,
                       bias_ref):
    @pl.when(i == 0)
    def _():
        wbot = wproj_ref[D:, :]
        wf_ref[...] = wprop_ref[...] @ wbot
        bias_ref[...] = bprop_ref[...] @ wbot + bproj_ref[...]


def _onehot_body(wproj_ref, wtop_ref, wprop_ref, bprop_ref, bproj_ref,
                 table_ref, ids_ref, prop_ref, out_ref, wf_ref, bias_ref):
    i = pl.program_id(0)
    _fuse_into_scratch(i, wproj_ref, wprop_ref, bprop_ref, bproj_ref, wf_ref,
                       bias_ref)
    common = prop_ref[...] @ wf_ref[...] + bias_ref[...]
    iota_v = lax.broadcasted_iota(jnp.int32, (V, 1), 0)
    oh_t = (ids_ref[0] == iota_v).astype(jnp.bfloat16)
    g_blk = lax.dot_general(oh_t, table_ref[...],
                            (((0,), (0,)), ((), ())),
                            preferred_element_type=jnp.float32)
    g_blk = g_blk.astype(jnp.bfloat16)
    out_ref[...] = lax.dot_general(g_blk, wtop_ref[...],
                                   (((1,), (0,)), ((), ())),
                                   preferred_element_type=jnp.float32) + common


def _gathered_body(prev_ref, wproj_ref, wtop_ref, wprop_ref, bprop_ref,
                   bproj_ref, g_ref, prop_ref, out_ref, wf_ref, bias_ref):
    del prev_ref
    i = pl.program_id(0)
    _fuse_into_scratch(i, wproj_ref, wprop_ref, bprop_ref, bproj_ref, wf_ref,
                       bias_ref)
    common = prop_ref[...] @ wf_ref[...] + bias_ref[...]
    g_bf = g_ref[...].astype(jnp.bfloat16)
    out_ref[...] = lax.dot_general(g_bf, wtop_ref[...],
                                   (((1,), (0,)), ((), ())),
                                   preferred_element_type=jnp.float32) + common


_WSPECS = [
    pl.BlockSpec((2 * D, D), lambda i: (0, 0)),
    pl.BlockSpec((D, D), lambda i: (0, 0)),
    pl.BlockSpec((P, D), lambda i: (0, 0)),
    pl.BlockSpec((1, D), lambda i: (0, 0)),
    pl.BlockSpec((1, D), lambda i: (0, 0)),
]
_SCRATCH = [
    pltpu.VMEM((P, D), jnp.float32),
    pltpu.VMEM((1, D), jnp.float32),
]
_OUT_SHAPE = jax.ShapeDtypeStruct((B, D), jnp.float32)


def _combine_onehot(w_proj, wtop_bf, w_prop, b_prop, b_proj, table_bf, ids3,
                    prop):
    return pl.pallas_call(
        _onehot_body,
        grid=(NB - NSCB,),
        in_specs=_WSPECS + [
            pl.BlockSpec((V, D), lambda i: (0, 0)),
            pl.BlockSpec((1, 1, BLK), lambda i: (i + NSCB, 0, 0)),
            pl.BlockSpec((BLK, P), lambda i: (i + NSCB, 0)),
        ],
        out_specs=pl.BlockSpec((BLK, D), lambda i: (i + NSCB, 0)),
        out_shape=_OUT_SHAPE,
        scratch_shapes=_SCRATCH,
    )(w_proj, wtop_bf, w_prop, b_prop, b_proj, table_bf, ids3, prop)


def _combine_gathered(prev, w_proj, wtop_bf, w_prop, b_prop, b_proj, g, prop):
    return pl.pallas_call(
        _gathered_body,
        grid=(NSCB,),
        in_specs=[pl.BlockSpec(memory_space=pl.ANY)] + _WSPECS + [
            pl.BlockSpec((BLK, D), lambda i: (i, 0)),
            pl.BlockSpec((BLK, P), lambda i: (i, 0)),
        ],
        out_specs=pl.BlockSpec((BLK, D), lambda i: (i, 0)),
        out_shape=_OUT_SHAPE,
        input_output_aliases={0: 0},
        scratch_shapes=_SCRATCH,
    )(prev, w_proj, wtop_bf, w_prop, b_prop, b_proj, g, prop)


def kernel(event_type_ids, prop_vectors, event_type_table, W_prop, b_prop,
           W_proj, b_proj):
    ids = event_type_ids.astype(jnp.int32)
    g = _make_sc_gather()(event_type_table, ids)
    bprop2 = b_prop.reshape(1, D)
    bproj2 = b_proj.reshape(1, D)
    ids3 = ids.reshape(NB, 1, BLK)
    wtop_bf = W_proj[:D].astype(jnp.bfloat16)
    table_bf = event_type_table.astype(jnp.bfloat16)
    out = _combine_onehot(W_proj, wtop_bf, W_prop, bprop2, bproj2, table_bf,
                          ids3, prop_vectors)
    return _combine_gathered(out, W_proj, wtop_bf, W_prop, bprop2, bproj2,
                             g, prop_vectors)


# speedup vs baseline: 1.1956x; 1.0004x over previous
import functools

import jax
import jax.numpy as jnp
from jax import lax
from jax.experimental import pallas as pl
from jax.experimental.pallas import tpu as pltpu
from jax.experimental.pallas import tpu_sc as plsc

B = 16384
D = 128
V = 100
VPAD = 128
P = 50

NC, NS = 2, 16
NW = NC * NS

B_SC = 8192
BPW = B_SC // NW

BLK = 2048
NB = B // BLK
NSCB = B_SC // BLK


@functools.cache
def _make_sc_gather():
    mesh = plsc.VectorSubcoreMesh(core_axis_name="c", subcore_axis_name="s",
                                  num_cores=NC, num_subcores=NS)

    @functools.partial(
        pl.kernel,
        out_type=jax.ShapeDtypeStruct((B_SC, D), jnp.float32),
        mesh=mesh,
        scratch_types=[
            pltpu.VMEM((BPW,), jnp.int32),
            pltpu.VMEM((BPW, D), jnp.float32),
            pltpu.SemaphoreType.DMA,
        ],
    )
    def _sc_gather(table_hbm, idx_hbm, out_hbm, idx_v, rows_v, sem):
        wid = lax.axis_index("s") * NC + lax.axis_index("c")
        base = wid * BPW
        pltpu.sync_copy(idx_hbm.at[pl.ds(base, BPW)], idx_v)
        pltpu.async_copy(table_hbm.at[idx_v], rows_v, sem).wait()
        pltpu.sync_copy(rows_v, out_hbm.at[pl.ds(base, BPW)])

    return _sc_gather


def _fuse_into_scratch(i, wproj_ref, wprop_ref, bprop_ref, bproj_ref, wf_ref,
                       bias_ref, wtop_bf_ref):
    @pl.when(i == 0)
    def _():
        wbot = wproj_ref[D:, :]
        wf_ref[...] = wprop_ref[...] @ wbot
        bias_ref[...] = bprop_ref[...] @ wbot + bproj_ref[...]
        wtop_bf_ref[...] = wproj_ref[:D, :].astype(jnp.bfloat16)


def _onehot_body(wproj_ref, wprop_ref, bprop_ref, bproj_ref,
                 table_ref, ids_ref, prop_ref, out_ref, wf_ref, bias_ref,
                 wtop_bf_ref, table_bf_ref):
    i = pl.program_id(0)
    _fuse_into_scratch(i, wproj_ref, wprop_ref, bprop_ref, bproj_ref, wf_ref,
                       bias_ref, wtop_bf_ref)

    @pl.when(i == 0)
    def _():
        table_bf_ref[...] = table_ref[...].astype(jnp.bfloat16)

    common = prop_ref[...] @ wf_ref[...] + bias_ref[...]
    iota_v = lax.broadcasted_iota(jnp.int32, (V, 1), 0)
    oh_t = (ids_ref[0] == iota_v).astype(jnp.bfloat16)
    g_blk = lax.dot_general(oh_t, table_bf_ref[...],
                            (((0,), (0,)), ((), ())),
                            preferred_element_type=jnp.float32)
    g_blk = g_blk.astype(jnp.bfloat16)
    out_ref[...] = lax.dot_general(g_blk, wtop_bf_ref[...],
                                   (((1,), (0,)), ((), ())),
                                   preferred_element_type=jnp.float32) + common


def _gathered_body(prev_ref, wproj_ref, wprop_ref, bprop_ref,
                   bproj_ref, g_ref, prop_ref, out_ref, wf_ref, bias_ref,
                   wtop_bf_ref):
    del prev_ref
    i = pl.program_id(0)
    _fuse_into_scratch(i, wproj_ref, wprop_ref, bprop_ref, bproj_ref, wf_ref,
                       bias_ref, wtop_bf_ref)
    common = prop_ref[...] @ wf_ref[...] + bias_ref[...]
    g_bf = g_ref[...].astype(jnp.bfloat16)
    out_ref[...] = lax.dot_general(g_bf, wtop_bf_ref[...],
                                   (((1,), (0,)), ((), ())),
                                   preferred_element_type=jnp.float32) + common


_WSPECS = [
    pl.BlockSpec((2 * D, D), lambda i: (0, 0)),
    pl.BlockSpec((P, D), lambda i: (0, 0)),
    pl.BlockSpec((1, D), lambda i: (0, 0)),
    pl.BlockSpec((1, D), lambda i: (0, 0)),
]
_SCRATCH = [
    pltpu.VMEM((P, D), jnp.float32),
    pltpu.VMEM((1, D), jnp.float32),
    pltpu.VMEM((D, D), jnp.bfloat16),
]
_OUT_SHAPE = jax.ShapeDtypeStruct((B, D), jnp.float32)


def _combine_onehot(w_proj, w_prop, b_prop, b_proj, table, ids3, prop_hi):
    return pl.pallas_call(
        _onehot_body,
        grid=(NB - NSCB,),
        in_specs=_WSPECS + [
            pl.BlockSpec((V, D), lambda i: (0, 0)),
            pl.BlockSpec((1, 1, BLK), lambda i: (i + NSCB, 0, 0)),
            pl.BlockSpec((BLK, P), lambda i: (i, 0)),
        ],
        out_specs=pl.BlockSpec((BLK, D), lambda i: (i + NSCB, 0)),
        out_shape=_OUT_SHAPE,
        scratch_shapes=_SCRATCH + [pltpu.VMEM((V, D), jnp.bfloat16)],
    )(w_proj, w_prop, b_prop, b_proj, table, ids3, prop_hi)


def _combine_gathered(prev, w_proj, w_prop, b_prop, b_proj, g, prop_lo):
    return pl.pallas_call(
        _gathered_body,
        grid=(NSCB,),
        in_specs=[pl.BlockSpec(memory_space=pl.ANY)] + _WSPECS + [
            pl.BlockSpec((BLK, D), lambda i: (i, 0)),
            pl.BlockSpec((BLK, P), lambda i: (i, 0)),
        ],
        out_specs=pl.BlockSpec((BLK, D), lambda i: (i, 0)),
        out_shape=_OUT_SHAPE,
        input_output_aliases={0: 0},
        scratch_shapes=_SCRATCH,
    )(prev, w_proj, w_prop, b_prop, b_proj, g, prop_lo)


def kernel(event_type_ids, prop_vectors, event_type_table, W_prop, b_prop,
           W_proj, b_proj):
    ids = event_type_ids.astype(jnp.int32)
    g = _make_sc_gather()(event_type_table, ids)
    bprop2 = b_prop.reshape(1, D)
    bproj2 = b_proj.reshape(1, D)
    ids3 = ids.reshape(NB, 1, BLK)
    prop_lo = prop_vectors[:B_SC]
    prop_hi = prop_vectors[B_SC:]
    out = _combine_onehot(W_proj, W_prop, bprop2, bproj2, event_type_table,
                          ids3, prop_hi)
    return _combine_gathered(out, W_proj, W_prop, bprop2, bproj2, g, prop_lo)


# speedup vs baseline: 1.2238x; 1.0236x over previous
import functools

import jax
import jax.numpy as jnp
from jax import lax
from jax.experimental import pallas as pl
from jax.experimental.pallas import tpu as pltpu
from jax.experimental.pallas import tpu_sc as plsc

B = 16384
D = 128
V = 100
P = 50

NC, NS = 2, 16
NW = NC * NS

B_SC = 8192
BPW = B_SC // NW

BLK = 2048
NB = B // BLK
NSCB = B_SC // BLK


@functools.cache
def _make_sc_gather():
    mesh = plsc.VectorSubcoreMesh(core_axis_name="c", subcore_axis_name="s",
                                  num_cores=NC, num_subcores=NS)

    @functools.partial(
        pl.kernel,
        out_type=jax.ShapeDtypeStruct((B_SC, D), jnp.float32),
        mesh=mesh,
        scratch_types=[
            pltpu.VMEM((BPW,), jnp.int32),
            pltpu.VMEM((BPW, D), jnp.float32),
            pltpu.SemaphoreType.DMA,
        ],
    )
    def _sc_gather(table_hbm, idx_hbm, out_hbm, idx_v, rows_v, sem):
        wid = lax.axis_index("s") * NC + lax.axis_index("c")
        base = wid * BPW
        pltpu.sync_copy(idx_hbm.at[pl.ds(base, BPW)], idx_v)
        pltpu.async_copy(table_hbm.at[idx_v], rows_v, sem).wait()
        pltpu.sync_copy(rows_v, out_hbm.at[pl.ds(base, BPW)])

    return _sc_gather


def _fuse_into_scratch(i, wproj_ref, wprop_ref, bprop_ref, bproj_ref, wf_ref,
                       bias_ref):
    @pl.when(i == 0)
    def _():
        wbot = wproj_ref[D:, :]
        wf_ref[...] = wprop_ref[...] @ wbot
        bias_ref[...] = bprop_ref[...] @ wbot + bproj_ref[...]


def _onehot_body(wproj_ref, wprop_ref, bprop_ref, bproj_ref, table_ref,
                 ids_ref, prop_ref, out_ref, wf_ref, bias_ref):
    i = pl.program_id(0)
    _fuse_into_scratch(i, wproj_ref, wprop_ref, bprop_ref, bproj_ref, wf_ref,
                       bias_ref)
    common = prop_ref[...] @ wf_ref[...] + bias_ref[...]
    iota_v = lax.broadcasted_iota(jnp.int32, (V, 1), 0)
    oh_t = (ids_ref[0] == iota_v).astype(jnp.float32)
    g_blk = lax.dot_general(oh_t, table_ref[...],
                            (((0,), (0,)), ((), ())))
    out_ref[...] = g_blk @ wproj_ref[:D, :] + common


def _gathered_body(prev_ref, wproj_ref, wprop_ref, bprop_ref, bproj_ref,
                   g_ref, prop_ref, out_ref, wf_ref, bias_ref):
    del prev_ref
    i = pl.program_id(0)
    _fuse_into_scratch(i, wproj_ref, wprop_ref, bprop_ref, bproj_ref, wf_ref,
                       bias_ref)
    common = prop_ref[...] @ wf_ref[...] + bias_ref[...]
    out_ref[...] = g_ref[...] @ wproj_ref[:D, :] + common


_WSPECS = [
    pl.BlockSpec((2 * D, D), lambda i: (0, 0)),
    pl.BlockSpec((P, D), lambda i: (0, 0)),
    pl.BlockSpec((1, D), lambda i: (0, 0)),
    pl.BlockSpec((1, D), lambda i: (0, 0)),
]
_SCRATCH = [
    pltpu.VMEM((P, D), jnp.float32),
    pltpu.VMEM((1, D), jnp.float32),
]
_OUT_SHAPE = jax.ShapeDtypeStruct((B, D), jnp.float32)


def _combine_onehot(w_proj, w_prop, b_prop, b_proj, table, ids3, prop):
    return pl.pallas_call(
        _onehot_body,
        grid=(NB - NSCB,),
        in_specs=_WSPECS + [
            pl.BlockSpec((V, D), lambda i: (0, 0)),
            pl.BlockSpec((1, 1, BLK), lambda i: (i + NSCB, 0, 0)),
            pl.BlockSpec((BLK, P), lambda i: (i + NSCB, 0)),
        ],
        out_specs=pl.BlockSpec((BLK, D), lambda i: (i + NSCB, 0)),
        out_shape=_OUT_SHAPE,
        scratch_shapes=_SCRATCH,
    )(w_proj, w_prop, b_prop, b_proj, table, ids3, prop)


def _combine_gathered(prev, w_proj, w_prop, b_prop, b_proj, g, prop):
    return pl.pallas_call(
        _gathered_body,
        grid=(NSCB,),
        in_specs=[pl.BlockSpec(memory_space=pl.ANY)] + _WSPECS + [
            pl.BlockSpec((BLK, D), lambda i: (i, 0)),
            pl.BlockSpec((BLK, P), lambda i: (i, 0)),
        ],
        out_specs=pl.BlockSpec((BLK, D), lambda i: (i, 0)),
        out_shape=_OUT_SHAPE,
        input_output_aliases={0: 0},
        scratch_shapes=_SCRATCH,
    )(prev, w_proj, w_prop, b_prop, b_proj, g, prop)


def kernel(event_type_ids, prop_vectors, event_type_table, W_prop, b_prop,
           W_proj, b_proj):
    ids = event_type_ids.astype(jnp.int32)
    g = _make_sc_gather()(event_type_table, ids)
    bprop2 = b_prop.reshape(1, D)
    bproj2 = b_proj.reshape(1, D)
    ids3 = ids.reshape(NB, 1, BLK)
    out = _combine_onehot(W_proj, W_prop, bprop2, bproj2, event_type_table,
                          ids3, prop_vectors)
    return _combine_gathered(out, W_proj, W_prop, bprop2, bproj2, g,
                             prop_vectors)
